# baseline (speedup 1.0000x reference)
import functools

import jax
import jax.numpy as jnp
from jax import lax
from jax.experimental import pallas as pl
from jax.experimental.pallas import tpu as pltpu
from jax.experimental.pallas import tpu_sc as plsc

NUM_USERS = 1000000
NUM_ITEMS = 1000000
FACTORS = 64
BATCH = 16384

NUM_CORES = 2
NUM_SUBCORES = 16
NW = NUM_CORES * NUM_SUBCORES
LANES = 16

BPW = BATCH // NW
WIN = 8
IDX_CHUNK = 128
NCHUNK = BPW // IDX_CHUNK

_MESH = plsc.VectorSubcoreMesh(core_axis_name="c", subcore_axis_name="s")


def _strip_one_table(row, sid_v, tabT_hbm, outT_hbm, strip_v, gt_v, sem, wid):
    fi = lax.iota(jnp.int32, LANES)

    def chunk(k, carry):
        cprev, sprev = carry
        ids16 = sid_v[row, pl.ds(k * LANES, LANES)]
        for half in range(2):
            ids = [ids16[half * WIN + j] for j in range(WIN)]
            cols = [i >> 7 for i in ids]
            offs = [i & 127 for i in ids]
            news, slots = [], []
            nd = jnp.int32(0)
            first_new = cols[0] != cprev
            sbase = jnp.where(first_new, jnp.int32(0), (sprev + 1) % WIN)
            for j in range(WIN):
                new = (cols[j] != cols[j - 1]) if j else first_new
                slot = jnp.where(new, (sbase + nd) % WIN, sprev)
                news.append(new)
                slots.append(slot)
                sprev = slot
                nd = nd + new.astype(jnp.int32)
            cprev = cols[WIN - 1]
            descs = [
                pltpu.make_async_copy(
                    tabT_hbm.at[:, pl.ds(cols[j] * 128, 128)],
                    strip_v.at[slots[j]], sem)
                for j in range(WIN)
            ]
            for j in range(WIN):
                @pl.when(news[j])
                def _(d=descs[j]):
                    d.start()
            for j in range(WIN):
                @pl.when(news[j])
                def _(d=descs[j]):
                    d.wait()
            for j in range(WIN):
                rloc = jnp.full((LANES,), k * LANES + half * WIN + j,
                                jnp.int32)
                coff = jnp.full((LANES,), 1, jnp.int32) * offs[j]
                for q in range(FACTORS // LANES):
                    x = plsc.load_gather(strip_v.at[slots[j]],
                                         [fi + q * LANES, coff])
                    plsc.store_scatter(gt_v, [fi + q * LANES, rloc], x)
        return (cprev, sprev)

    lax.fori_loop(0, BPW // LANES, chunk, (jnp.int32(-1), jnp.int32(0)))
    pltpu.sync_copy(gt_v, outT_hbm.at[:, pl.ds(wid * BPW, BPW)])


def _strip_body(su_hbm, si_hbm, guT_hbm, giT_hbm, gsuT_hbm, gsiT_hbm,
                su_v, si_v, strip_v, gt_v, sem):
    wid = lax.axis_index("s") * NUM_CORES + lax.axis_index("c")
    pltpu.sync_copy(su_hbm.at[pl.ds((wid // 8) * 8, 8)], su_v)
    pltpu.sync_copy(si_hbm.at[pl.ds((wid // 8) * 8, 8)], si_v)
    row = wid % 8
    _strip_one_table(row, su_v, guT_hbm, gsuT_hbm, strip_v, gt_v, sem, wid)
    _strip_one_table(row, si_v, giT_hbm, gsiT_hbm, strip_v, gt_v, sem, wid)


@functools.partial(
    pl.kernel,
    out_type=(
        jax.ShapeDtypeStruct((FACTORS, BATCH), jnp.float32),
        jax.ShapeDtypeStruct((FACTORS, BATCH), jnp.float32),
    ),
    mesh=_MESH,
    compiler_params=pltpu.CompilerParams(
        use_tc_tiling_on_sc=True, needs_layout_passes=False),
    scratch_types=[
        pltpu.VMEM((8, 512), jnp.int32),
        pltpu.VMEM((8, 512), jnp.int32),
        pltpu.VMEM((WIN, FACTORS, 128), jnp.float32),
        pltpu.VMEM((FACTORS, BPW), jnp.float32),
        pltpu.SemaphoreType.DMA,
    ],
)
def _strip_gather(su_hbm, si_hbm, guT_hbm, giT_hbm, *rest):
    _strip_body(su_hbm, si_hbm, guT_hbm, giT_hbm, *rest)


def _unsort_body(gsu_hbm, gsi_hbm, pu_hbm, qi_hbm, si_hbm, bi_hbm,
                 gu_hbm, gi_hbm, beta_hbm,
                 pu_v, qi_v, si_v, rows_u, rows_i, bi_v,
                 sem_i, sem_u, sem_g, sem_b):
    wid = lax.axis_index("s") * NUM_CORES + lax.axis_index("c")
    base = wid * BPW
    c_pu = pltpu.async_copy(pu_hbm.at[pl.ds(wid * NCHUNK, NCHUNK)], pu_v, sem_i)
    c_qi = pltpu.async_copy(qi_hbm.at[pl.ds(wid * NCHUNK, NCHUNK)], qi_v, sem_i)
    c_si = pltpu.async_copy(si_hbm.at[pl.ds(wid * NCHUNK, NCHUNK)], si_v, sem_i)
    c_ru = pltpu.async_copy(gsu_hbm.at[pl.ds(base, BPW)], rows_u, sem_u)
    c_ri = pltpu.async_copy(gsi_hbm.at[pl.ds(base, BPW)], rows_i, sem_u)
    c_pu.wait(); c_qi.wait(); c_si.wait()
    bcs = [pltpu.async_copy(bi_hbm.at[si_v.at[j]],
                            bi_v.at[pl.ds(j * IDX_CHUNK, IDX_CHUNK)], sem_b)
           for j in range(NCHUNK)]
    c_ru.wait()
    ucs = [pltpu.async_copy(rows_u.at[pl.ds(j * IDX_CHUNK, IDX_CHUNK)],
                            gu_hbm.at[pu_v.at[j]], sem_g)
           for j in range(NCHUNK)]
    c_ri.wait()
    ics = [pltpu.async_copy(rows_i.at[pl.ds(j * IDX_CHUNK, IDX_CHUNK)],
                            gi_hbm.at[qi_v.at[j]], sem_g)
           for j in range(NCHUNK)]
    for cp in bcs:
        cp.wait()
    bss = [pltpu.async_copy(bi_v.at[pl.ds(j * IDX_CHUNK, IDX_CHUNK)],
                            beta_hbm.at[qi_v.at[j]], sem_b)
           for j in range(NCHUNK)]
    for cp in ucs:
        cp.wait()
    for cp in ics:
        cp.wait()
    for cp in bss:
        cp.wait()


@functools.partial(
    pl.kernel,
    out_type=(
        jax.ShapeDtypeStruct((BATCH, FACTORS), jnp.float32),
        jax.ShapeDtypeStruct((BATCH, FACTORS), jnp.float32),
        jax.ShapeDtypeStruct((BATCH,), jnp.float32),
    ),
    mesh=_MESH,
    compiler_params=pltpu.CompilerParams(use_tc_tiling_on_sc=False),
    scratch_types=[
        pltpu.VMEM((NCHUNK, IDX_CHUNK), jnp.int32),
        pltpu.VMEM((NCHUNK, IDX_CHUNK), jnp.int32),
        pltpu.VMEM((NCHUNK, IDX_CHUNK), jnp.int32),
        pltpu.VMEM((BPW, FACTORS), jnp.float32),
        pltpu.VMEM((BPW, FACTORS), jnp.float32),
        pltpu.VMEM((BPW,), jnp.float32),
        pltpu.SemaphoreType.DMA,
        pltpu.SemaphoreType.DMA,
        pltpu.SemaphoreType.DMA,
        pltpu.SemaphoreType.DMA,
    ],
)
def _unsort(gsu_hbm, gsi_hbm, pu_hbm, qi_hbm, si_hbm, bi_hbm, *rest):
    _unsort_body(gsu_hbm, gsi_hbm, pu_hbm, qi_hbm, si_hbm, bi_hbm, *rest)


TC_BLOCK = 2048


def _dot_tc_body(gu_ref, gi_ref, beta_ref, xui_ref):
    xui_ref[...] = beta_ref[...] + jnp.sum(gu_ref[...] * gi_ref[...], axis=1)


_dot_tc = pl.pallas_call(
    _dot_tc_body,
    grid=(BATCH // TC_BLOCK,),
    in_specs=[
        pl.BlockSpec((TC_BLOCK, FACTORS), lambda i: (i, 0)),
        pl.BlockSpec((TC_BLOCK, FACTORS), lambda i: (i, 0)),
        pl.BlockSpec((TC_BLOCK,), lambda i: (i,)),
    ],
    out_specs=pl.BlockSpec((TC_BLOCK,), lambda i: (i,)),
    out_shape=jax.ShapeDtypeStruct((BATCH,), jnp.float32),
)


def kernel(users_indices, items_indices, Gu, Gi, Bi):
    iu = users_indices.astype(jnp.int32)
    ii = items_indices.astype(jnp.int32)
    pos = lax.iota(jnp.int32, BATCH)
    su, pu = lax.sort((iu, pos), num_keys=1)
    si, qi = lax.sort((ii, pos), num_keys=1)
    gsuT, gsiT = _strip_gather(su.reshape(NW, BPW), si.reshape(NW, BPW),
                               Gu.T, Gi.T)
    gamma_u, gamma_i, beta_i = _unsort(
        gsuT.T, gsiT.T,
        pu.reshape(NW * NCHUNK, IDX_CHUNK),
        qi.reshape(NW * NCHUNK, IDX_CHUNK),
        si.reshape(NW * NCHUNK, IDX_CHUNK),
        Bi.reshape(NUM_ITEMS))
    xui = _dot_tc(gamma_u, gamma_i, beta_i)
    return (xui, beta_i, gamma_u, gamma_i)


# speedup vs baseline: 1.0194x; 1.0194x over previous
import functools

import jax
import jax.numpy as jnp
from jax import lax
from jax.experimental import pallas as pl
from jax.experimental.pallas import tpu as pltpu
from jax.experimental.pallas import tpu_sc as plsc

NUM_USERS = 1000000
NUM_ITEMS = 1000000
FACTORS = 64
BATCH = 16384

NUM_CORES = 2
NUM_SUBCORES = 16
NW = NUM_CORES * NUM_SUBCORES
LANES = 16

BPW = BATCH // NW
WIN = 8
IDX_CHUNK = 128
NCHUNK = BPW // IDX_CHUNK

_MESH = plsc.VectorSubcoreMesh(core_axis_name="c", subcore_axis_name="s")


def _strip_one_table(row, sid_v, tabT_hbm, outT_hbm, strip_v, gt_v, sem, wid):
    fi = lax.iota(jnp.int32, LANES)

    def chunk(k, carry):
        cprev, sprev = carry
        ids16 = sid_v[row, pl.ds(k * LANES, LANES)]
        for half in range(2):
            ids = [ids16[half * WIN + j] for j in range(WIN)]
            cols = [i >> 7 for i in ids]
            offs = [i & 127 for i in ids]
            news, slots = [], []
            nd = jnp.int32(0)
            first_new = cols[0] != cprev
            sbase = jnp.where(first_new, jnp.int32(0), (sprev + 1) % WIN)
            for j in range(WIN):
                new = (cols[j] != cols[j - 1]) if j else first_new
                slot = jnp.where(new, (sbase + nd) % WIN, sprev)
                news.append(new)
                slots.append(slot)
                sprev = slot
                nd = nd + new.astype(jnp.int32)
            cprev = cols[WIN - 1]
            descs = [
                pltpu.make_async_copy(
                    tabT_hbm.at[:, pl.ds(cols[j] * 128, 128)],
                    strip_v.at[slots[j]], sem)
                for j in range(WIN)
            ]
            for j in range(WIN):
                @pl.when(news[j])
                def _(d=descs[j]):
                    d.start()
            for j in range(WIN):
                @pl.when(news[j])
                def _(d=descs[j]):
                    d.wait()
            for j in range(WIN):
                rloc = jnp.full((LANES,), k * LANES + half * WIN + j,
                                jnp.int32)
                coff = jnp.full((LANES,), 1, jnp.int32) * offs[j]
                for q in range(FACTORS // LANES):
                    x = plsc.load_gather(strip_v.at[slots[j]],
                                         [fi + q * LANES, coff])
                    plsc.store_scatter(gt_v, [fi + q * LANES, rloc], x)
        return (cprev, sprev)

    lax.fori_loop(0, BPW // LANES, chunk, (jnp.int32(-1), jnp.int32(0)))
    pltpu.sync_copy(gt_v, outT_hbm.at[:, pl.ds(wid * BPW, BPW)])


def _strip_body(sids_hbm, tabT_hbm, outT_hbm, sid_v, strip_v, gt_v, sem):
    wid = lax.axis_index("s") * NUM_CORES + lax.axis_index("c")
    pltpu.sync_copy(sids_hbm.at[pl.ds((wid // 8) * 8, 8)], sid_v)
    row = wid % 8
    _strip_one_table(row, sid_v, tabT_hbm, outT_hbm, strip_v, gt_v, sem, wid)


@functools.partial(
    pl.kernel,
    out_type=jax.ShapeDtypeStruct((FACTORS, BATCH), jnp.float32),
    mesh=_MESH,
    compiler_params=pltpu.CompilerParams(
        use_tc_tiling_on_sc=True, needs_layout_passes=False),
    scratch_types=[
        pltpu.VMEM((8, 512), jnp.int32),
        pltpu.VMEM((WIN, FACTORS, 128), jnp.float32),
        pltpu.VMEM((FACTORS, BPW), jnp.float32),
        pltpu.SemaphoreType.DMA,
    ],
)
def _strip_gather(sids_hbm, tabT_hbm, *rest):
    _strip_body(sids_hbm, tabT_hbm, *rest)


def _unsort_body(gsu_hbm, gsi_hbm, pu_hbm, qi_hbm, si_hbm, bi_hbm,
                 gu_hbm, gi_hbm, beta_hbm,
                 pu_v, qi_v, si_v, rows_u, rows_i, bi_v,
                 sem_i, sem_u, sem_g, sem_b):
    wid = lax.axis_index("s") * NUM_CORES + lax.axis_index("c")
    base = wid * BPW
    c_pu = pltpu.async_copy(pu_hbm.at[pl.ds(wid * NCHUNK, NCHUNK)], pu_v, sem_i)
    c_qi = pltpu.async_copy(qi_hbm.at[pl.ds(wid * NCHUNK, NCHUNK)], qi_v, sem_i)
    c_si = pltpu.async_copy(si_hbm.at[pl.ds(wid * NCHUNK, NCHUNK)], si_v, sem_i)
    c_ru = pltpu.async_copy(gsu_hbm.at[pl.ds(base, BPW)], rows_u, sem_u)
    c_ri = pltpu.async_copy(gsi_hbm.at[pl.ds(base, BPW)], rows_i, sem_u)
    c_pu.wait(); c_qi.wait(); c_si.wait()
    bcs = [pltpu.async_copy(bi_hbm.at[si_v.at[j]],
                            bi_v.at[pl.ds(j * IDX_CHUNK, IDX_CHUNK)], sem_b)
           for j in range(NCHUNK)]
    c_ru.wait()
    ucs = [pltpu.async_copy(rows_u.at[pl.ds(j * IDX_CHUNK, IDX_CHUNK)],
                            gu_hbm.at[pu_v.at[j]], sem_g)
           for j in range(NCHUNK)]
    c_ri.wait()
    ics = [pltpu.async_copy(rows_i.at[pl.ds(j * IDX_CHUNK, IDX_CHUNK)],
                            gi_hbm.at[qi_v.at[j]], sem_g)
           for j in range(NCHUNK)]
    for cp in bcs:
        cp.wait()
    bss = [pltpu.async_copy(bi_v.at[pl.ds(j * IDX_CHUNK, IDX_CHUNK)],
                            beta_hbm.at[qi_v.at[j]], sem_b)
           for j in range(NCHUNK)]
    for cp in ucs:
        cp.wait()
    for cp in ics:
        cp.wait()
    for cp in bss:
        cp.wait()


@functools.partial(
    pl.kernel,
    out_type=(
        jax.ShapeDtypeStruct((BATCH, FACTORS), jnp.float32),
        jax.ShapeDtypeStruct((BATCH, FACTORS), jnp.float32),
        jax.ShapeDtypeStruct((BATCH,), jnp.float32),
    ),
    mesh=_MESH,
    compiler_params=pltpu.CompilerParams(use_tc_tiling_on_sc=False),
    scratch_types=[
        pltpu.VMEM((NCHUNK, IDX_CHUNK), jnp.int32),
        pltpu.VMEM((NCHUNK, IDX_CHUNK), jnp.int32),
        pltpu.VMEM((NCHUNK, IDX_CHUNK), jnp.int32),
        pltpu.VMEM((BPW, FACTORS), jnp.float32),
        pltpu.VMEM((BPW, FACTORS), jnp.float32),
        pltpu.VMEM((BPW,), jnp.float32),
        pltpu.SemaphoreType.DMA,
        pltpu.SemaphoreType.DMA,
        pltpu.SemaphoreType.DMA,
        pltpu.SemaphoreType.DMA,
    ],
)
def _unsort(gsu_hbm, gsi_hbm, pu_hbm, qi_hbm, si_hbm, bi_hbm, *rest):
    _unsort_body(gsu_hbm, gsi_hbm, pu_hbm, qi_hbm, si_hbm, bi_hbm, *rest)


TC_BLOCK = 2048


def _dot_tc_body(gu_ref, gi_ref, beta_ref, xui_ref):
    xui_ref[...] = beta_ref[...] + jnp.sum(gu_ref[...] * gi_ref[...], axis=1)


_dot_tc = pl.pallas_call(
    _dot_tc_body,
    grid=(BATCH // TC_BLOCK,),
    in_specs=[
        pl.BlockSpec((TC_BLOCK, FACTORS), lambda i: (i, 0)),
        pl.BlockSpec((TC_BLOCK, FACTORS), lambda i: (i, 0)),
        pl.BlockSpec((TC_BLOCK,), lambda i: (i,)),
    ],
    out_specs=pl.BlockSpec((TC_BLOCK,), lambda i: (i,)),
    out_shape=jax.ShapeDtypeStruct((BATCH,), jnp.float32),
)


def kernel(users_indices, items_indices, Gu, Gi, Bi):
    iu = users_indices.astype(jnp.int32)
    ii = items_indices.astype(jnp.int32)
    pos = lax.iota(jnp.int32, BATCH)
    su, pu = lax.sort((iu, pos), num_keys=1)
    si, qi = lax.sort((ii, pos), num_keys=1)
    gsuT = _strip_gather(su.reshape(NW, BPW), Gu.T)
    gsiT = _strip_gather(si.reshape(NW, BPW), Gi.T)
    gamma_u, gamma_i, beta_i = _unsort(
        gsuT.T, gsiT.T,
        pu.reshape(NW * NCHUNK, IDX_CHUNK),
        qi.reshape(NW * NCHUNK, IDX_CHUNK),
        si.reshape(NW * NCHUNK, IDX_CHUNK),
        Bi.reshape(NUM_ITEMS))
    xui = _dot_tc(gamma_u, gamma_i, beta_i)
    return (xui, beta_i, gamma_u, gamma_i)


# speedup vs baseline: 1.2490x; 1.2252x over previous
import functools

import jax
import jax.numpy as jnp
from jax import lax
from jax.experimental import pallas as pl
from jax.experimental.pallas import tpu as pltpu
from jax.experimental.pallas import tpu_sc as plsc

NUM_USERS = 1000000
NUM_ITEMS = 1000000
FACTORS = 64
BATCH = 16384

NUM_CORES = 2
NUM_SUBCORES = 16
NW = NUM_CORES * NUM_SUBCORES
LANES = 16

BPW = BATCH // NW
PWIN = 4
BANK = 5
NSLOT = 2 * BANK
IDX_CHUNK = 128
NCHUNK = BPW // IDX_CHUNK

_MESH = plsc.VectorSubcoreMesh(core_axis_name="c", subcore_axis_name="s")


def _strip_one_table(row, sid_v, tabT_hbm, outT_hbm, strip_v, gt_v, sem, wid):
    fi = lax.iota(jnp.int32, LANES)
    z = jnp.int32(0)
    neg1 = jnp.int32(-1)

    def extract(slots, offs, rbase):
        for j in range(PWIN):
            rloc = jnp.full((LANES,), 1, jnp.int32) * (rbase + j)
            coff = jnp.full((LANES,), 1, jnp.int32) * offs[j]
            for q in range(FACTORS // LANES):
                x = plsc.load_gather(strip_v.at[slots[j]],
                                     [fi + q * LANES, coff])
                plsc.store_scatter(gt_v, [fi + q * LANES, rloc], x)

    def drain_one(_, c):
        pltpu.make_async_copy(
            tabT_hbm.at[:, pl.ds(0, 128)], strip_v.at[0], sem).wait()
        return c

    def do_window(w, sub, ids16, st):
        (cprev, sprev, age, prot, ndp, ps, po) = st
        ids = [ids16[sub * PWIN + j] for j in range(PWIN)]
        cols = [i >> 7 for i in ids]
        offs = [i & 127 for i in ids]
        bank = (w % 2) * BANK
        cont = cols[0] == cprev
        reuse = jnp.logical_and(cont, age == 0)
        news, slots = [], []
        pos = z
        nd = z
        sp = sprev
        for j in range(PWIN):
            new = (cols[j] != cols[j - 1]) if j else jnp.logical_not(reuse)
            cand = bank + pos
            hit = cand == prot
            cand = jnp.where(hit, cand + 1, cand)
            posn = pos + hit.astype(jnp.int32)
            slot = jnp.where(new, cand, sp)
            pos = jnp.where(new, posn + 1, pos)
            nd = nd + new.astype(jnp.int32)
            news.append(new)
            slots.append(slot)
            sp = slot
        descs = [
            pltpu.make_async_copy(
                tabT_hbm.at[:, pl.ds(cols[j] * 128, 128)],
                strip_v.at[slots[j]], sem)
            for j in range(PWIN)
        ]
        for j in range(PWIN):
            @pl.when(news[j])
            def _(d=descs[j]):
                d.start()
        lax.fori_loop(0, ndp, drain_one, 0)

        @pl.when(w > 0)
        def _():
            extract(ps, po, (w - 1) * PWIN)
        age_next = jnp.where(nd == 0, age + 1, z)
        prot_next = jnp.where(reuse, sprev, neg1)
        return (cols[PWIN - 1], sp, age_next, prot_next, nd, slots, offs)

    def chunk(k, carry):
        (cprev, sprev, age, prot, ndp,
         a0, a1, a2, a3, b0, b1, b2, b3) = carry
        st = (cprev, sprev, age, prot, ndp, [a0, a1, a2, a3],
              [b0, b1, b2, b3])
        ids16 = sid_v[row, pl.ds(k * LANES, LANES)]
        for sub in range(LANES // PWIN):
            st = do_window(k * (LANES // PWIN) + sub, sub, ids16, st)
        (cprev, sprev, age, prot, ndp, ps, po) = st
        return (cprev, sprev, age, prot, ndp,
                ps[0], ps[1], ps[2], ps[3], po[0], po[1], po[2], po[3])

    nwin = BPW // PWIN
    carry = lax.fori_loop(
        0, BPW // LANES, chunk,
        (neg1, z, jnp.int32(9), neg1, z, z, z, z, z, z, z, z, z))
    (_, _, _, _, ndp, a0, a1, a2, a3, b0, b1, b2, b3) = carry
    lax.fori_loop(0, ndp, drain_one, 0)
    extract([a0, a1, a2, a3], [b0, b1, b2, b3], (nwin - 1) * PWIN)
    pltpu.sync_copy(gt_v, outT_hbm.at[:, pl.ds(wid * BPW, BPW)])


def _strip_body(sids_hbm, tabT_hbm, outT_hbm, sid_v, strip_v, gt_v, sem):
    wid = lax.axis_index("s") * NUM_CORES + lax.axis_index("c")
    pltpu.sync_copy(sids_hbm.at[pl.ds((wid // 8) * 8, 8)], sid_v)
    row = wid % 8
    _strip_one_table(row, sid_v, tabT_hbm, outT_hbm, strip_v, gt_v, sem, wid)


@functools.partial(
    pl.kernel,
    out_type=jax.ShapeDtypeStruct((FACTORS, BATCH), jnp.float32),
    mesh=_MESH,
    compiler_params=pltpu.CompilerParams(
        use_tc_tiling_on_sc=True, needs_layout_passes=False),
    scratch_types=[
        pltpu.VMEM((8, 512), jnp.int32),
        pltpu.VMEM((NSLOT, FACTORS, 128), jnp.float32),
        pltpu.VMEM((FACTORS, BPW), jnp.float32),
        pltpu.SemaphoreType.DMA,
    ],
)
def _strip_gather(sids_hbm, tabT_hbm, *rest):
    _strip_body(sids_hbm, tabT_hbm, *rest)


def _unsort_body(gsu_hbm, gsi_hbm, pu_hbm, qi_hbm, si_hbm, bi_hbm,
                 gu_hbm, gi_hbm, beta_hbm,
                 pu_v, qi_v, si_v, rows_u, rows_i, bi_v,
                 sem_i, sem_u, sem_g, sem_b):
    wid = lax.axis_index("s") * NUM_CORES + lax.axis_index("c")
    base = wid * BPW
    c_pu = pltpu.async_copy(pu_hbm.at[pl.ds(wid * NCHUNK, NCHUNK)], pu_v, sem_i)
    c_qi = pltpu.async_copy(qi_hbm.at[pl.ds(wid * NCHUNK, NCHUNK)], qi_v, sem_i)
    c_si = pltpu.async_copy(si_hbm.at[pl.ds(wid * NCHUNK, NCHUNK)], si_v, sem_i)
    c_ru = pltpu.async_copy(gsu_hbm.at[pl.ds(base, BPW)], rows_u, sem_u)
    c_ri = pltpu.async_copy(gsi_hbm.at[pl.ds(base, BPW)], rows_i, sem_u)
    c_pu.wait(); c_qi.wait(); c_si.wait()
    bcs = [pltpu.async_copy(bi_hbm.at[si_v.at[j]],
                            bi_v.at[pl.ds(j * IDX_CHUNK, IDX_CHUNK)], sem_b)
           for j in range(NCHUNK)]
    c_ru.wait()
    ucs = [pltpu.async_copy(rows_u.at[pl.ds(j * IDX_CHUNK, IDX_CHUNK)],
                            gu_hbm.at[pu_v.at[j]], sem_g)
           for j in range(NCHUNK)]
    c_ri.wait()
    ics = [pltpu.async_copy(rows_i.at[pl.ds(j * IDX_CHUNK, IDX_CHUNK)],
                            gi_hbm.at[qi_v.at[j]], sem_g)
           for j in range(NCHUNK)]
    for cp in bcs:
        cp.wait()
    bss = [pltpu.async_copy(bi_v.at[pl.ds(j * IDX_CHUNK, IDX_CHUNK)],
                            beta_hbm.at[qi_v.at[j]], sem_b)
           for j in range(NCHUNK)]
    for cp in ucs:
        cp.wait()
    for cp in ics:
        cp.wait()
    for cp in bss:
        cp.wait()


@functools.partial(
    pl.kernel,
    out_type=(
        jax.ShapeDtypeStruct((BATCH, FACTORS), jnp.float32),
        jax.ShapeDtypeStruct((BATCH, FACTORS), jnp.float32),
        jax.ShapeDtypeStruct((BATCH,), jnp.float32),
    ),
    mesh=_MESH,
    compiler_params=pltpu.CompilerParams(use_tc_tiling_on_sc=False),
    scratch_types=[
        pltpu.VMEM((NCHUNK, IDX_CHUNK), jnp.int32),
        pltpu.VMEM((NCHUNK, IDX_CHUNK), jnp.int32),
        pltpu.VMEM((NCHUNK, IDX_CHUNK), jnp.int32),
        pltpu.VMEM((BPW, FACTORS), jnp.float32),
        pltpu.VMEM((BPW, FACTORS), jnp.float32),
        pltpu.VMEM((BPW,), jnp.float32),
        pltpu.SemaphoreType.DMA,
        pltpu.SemaphoreType.DMA,
        pltpu.SemaphoreType.DMA,
        pltpu.SemaphoreType.DMA,
    ],
)
def _unsort(gsu_hbm, gsi_hbm, pu_hbm, qi_hbm, si_hbm, bi_hbm, *rest):
    _unsort_body(gsu_hbm, gsi_hbm, pu_hbm, qi_hbm, si_hbm, bi_hbm, *rest)


TC_BLOCK = 2048


def _dot_tc_body(gu_ref, gi_ref, beta_ref, xui_ref):
    xui_ref[...] = beta_ref[...] + jnp.sum(gu_ref[...] * gi_ref[...], axis=1)


_dot_tc = pl.pallas_call(
    _dot_tc_body,
    grid=(BATCH // TC_BLOCK,),
    in_specs=[
        pl.BlockSpec((TC_BLOCK, FACTORS), lambda i: (i, 0)),
        pl.BlockSpec((TC_BLOCK, FACTORS), lambda i: (i, 0)),
        pl.BlockSpec((TC_BLOCK,), lambda i: (i,)),
    ],
    out_specs=pl.BlockSpec((TC_BLOCK,), lambda i: (i,)),
    out_shape=jax.ShapeDtypeStruct((BATCH,), jnp.float32),
)


def kernel(users_indices, items_indices, Gu, Gi, Bi):
    iu = users_indices.astype(jnp.int32)
    ii = items_indices.astype(jnp.int32)
    pos = lax.iota(jnp.int32, BATCH)
    su, pu = lax.sort((iu, pos), num_keys=1)
    si, qi = lax.sort((ii, pos), num_keys=1)
    gsuT = _strip_gather(su.reshape(NW, BPW), Gu.T)
    gsiT = _strip_gather(si.reshape(NW, BPW), Gi.T)
    gamma_u, gamma_i, beta_i = _unsort(
        gsuT.T, gsiT.T,
        pu.reshape(NW * NCHUNK, IDX_CHUNK),
        qi.reshape(NW * NCHUNK, IDX_CHUNK),
        si.reshape(NW * NCHUNK, IDX_CHUNK),
        Bi.reshape(NUM_ITEMS))
    xui = _dot_tc(gamma_u, gamma_i, beta_i)
    return (xui, beta_i, gamma_u, gamma_i)
